# Initial kernel scaffold; baseline (speedup 1.0000x reference)
import jax
import jax.numpy as jnp
from jax.experimental import pallas as pl


def kernel(nodes_features, nodes_neighbors_indexes, W, b, gamma, beta):
    raise NotImplementedError("write your pallas kernel here")


# speedup vs baseline: 1.5001x; 1.5001x over previous
import functools

import jax
import jax.numpy as jnp
from jax import lax
from jax.experimental import pallas as pl
from jax.experimental.pallas import tpu as pltpu
from jax.experimental.pallas import tpu_sc as plsc

N = 10000
M = 256
K = 16
OUT = 512

NC, NS = 2, 16
NW = NC * NS
CHUNK = 160
PER_W = 320
N_PAD = NW * PER_W
DV = M // 16


def _sc_body(x_hbm, idxT_hbm, h_hbm, idx_v, buf0, buf1, acc_v, sem0, sem1):
    wid = lax.axis_index("s") * NC + lax.axis_index("c")
    base = wid * PER_W
    bufs = (buf0, buf1)
    sems = (sem0, sem1)

    for chunk in range(PER_W // CHUNK):
        nbase = base + chunk * CHUNK
        for k in range(K):
            pltpu.sync_copy(idxT_hbm.at[pl.ds(k * N_PAD + nbase, CHUNK)],
                            idx_v.at[pl.ds(k * CHUNK, CHUNK)])
        cps = [None, None]
        cps[0] = pltpu.async_copy(
            x_hbm.at[idx_v.at[pl.ds(0, CHUNK)]], bufs[0], sems[0])
        for k in range(K):
            if k + 1 < K:
                cps[(k + 1) % 2] = pltpu.async_copy(
                    x_hbm.at[idx_v.at[pl.ds((k + 1) * CHUNK, CHUNK)]],
                    bufs[(k + 1) % 2], sems[(k + 1) % 2])
            cps[k % 2].wait()
            buf = bufs[k % 2]
            if k == 0:
                @pl.loop(0, CHUNK)
                def _init(r):
                    for dv in range(DV):
                        acc_v[r, pl.ds(dv * 16, 16)] = buf[r, pl.ds(dv * 16, 16)]
            else:
                @pl.loop(0, CHUNK)
                def _accum(r):
                    for dv in range(DV):
                        plsc.addupdate(acc_v.at[r, pl.ds(dv * 16, 16)],
                                       buf[r, pl.ds(dv * 16, 16)])
        pltpu.sync_copy(acc_v, h_hbm.at[pl.ds(nbase, CHUNK)])


def _neighbor_sum(x, idxT_pad):
    kfn = pl.kernel(
        _sc_body,
        out_type=jax.ShapeDtypeStruct((N_PAD, M), jnp.float32),
        mesh=plsc.VectorSubcoreMesh(core_axis_name="c", subcore_axis_name="s"),
        scratch_types=[
            pltpu.VMEM((K * CHUNK,), jnp.int32),
            pltpu.VMEM((CHUNK, M), jnp.float32),
            pltpu.VMEM((CHUNK, M), jnp.float32),
            pltpu.VMEM((CHUNK, M), jnp.float32),
            pltpu.SemaphoreType.DMA,
            pltpu.SemaphoreType.DMA,
        ],
    )
    return kfn(x, idxT_pad)


ROWS = 1000
GRID = N // ROWS


def _tc_matmul_body(x_ref, hnb_ref, wt_ref, b_ref, y_ref, s_ref, s2_ref):
    i = pl.program_id(0)
    h = x_ref[...] + hnb_ref[...]
    y = jnp.dot(h, wt_ref[...], preferred_element_type=jnp.float32) + b_ref[...]
    y_ref[...] = y
    s = jnp.sum(y, axis=0, keepdims=True)
    s2 = jnp.sum(y * y, axis=0, keepdims=True)

    @pl.when(i == 0)
    def _():
        s_ref[...] = s
        s2_ref[...] = s2

    @pl.when(i > 0)
    def _():
        s_ref[...] += s
        s2_ref[...] += s2


def _tc_bn_body(y_ref, s_ref, s2_ref, g_ref, beta_ref, o_ref):
    mean = s_ref[...] * (1.0 / N)
    var = s2_ref[...] * (1.0 / N) - mean * mean
    scale = g_ref[...] * lax.rsqrt(var + 1e-5)
    shift = beta_ref[...] - mean * scale
    o_ref[...] = jnp.maximum(y_ref[...] * scale + shift, 0.0)


def kernel(nodes_features, nodes_neighbors_indexes, W, b, gamma, beta):
    x = nodes_features
    idxT_pad = jnp.pad(nodes_neighbors_indexes.T,
                       ((0, 0), (0, N_PAD - N))).reshape(-1)
    hnb = _neighbor_sum(x, idxT_pad)[:N]

    wt = W.T
    b2 = b.reshape(1, OUT)
    g2 = gamma.reshape(1, OUT)
    beta2 = beta.reshape(1, OUT)

    y, s, s2 = pl.pallas_call(
        _tc_matmul_body,
        grid=(GRID,),
        in_specs=[
            pl.BlockSpec((ROWS, M), lambda i: (i, 0)),
            pl.BlockSpec((ROWS, M), lambda i: (i, 0)),
            pl.BlockSpec((M, OUT), lambda i: (0, 0)),
            pl.BlockSpec((1, OUT), lambda i: (0, 0)),
        ],
        out_specs=[
            pl.BlockSpec((ROWS, OUT), lambda i: (i, 0)),
            pl.BlockSpec((1, OUT), lambda i: (0, 0)),
            pl.BlockSpec((1, OUT), lambda i: (0, 0)),
        ],
        out_shape=[
            jax.ShapeDtypeStruct((N, OUT), jnp.float32),
            jax.ShapeDtypeStruct((1, OUT), jnp.float32),
            jax.ShapeDtypeStruct((1, OUT), jnp.float32),
        ],
    )(x, hnb, wt, b2)

    out = pl.pallas_call(
        _tc_bn_body,
        grid=(GRID,),
        in_specs=[
            pl.BlockSpec((ROWS, OUT), lambda i: (i, 0)),
            pl.BlockSpec((1, OUT), lambda i: (0, 0)),
            pl.BlockSpec((1, OUT), lambda i: (0, 0)),
            pl.BlockSpec((1, OUT), lambda i: (0, 0)),
            pl.BlockSpec((1, OUT), lambda i: (0, 0)),
        ],
        out_specs=pl.BlockSpec((ROWS, OUT), lambda i: (i, 0)),
        out_shape=jax.ShapeDtypeStruct((N, OUT), jnp.float32),
    )(y, s, s2, g2, beta2)

    return (out, nodes_neighbors_indexes)
